# Initial kernel scaffold; baseline (speedup 1.0000x reference)
import jax
import jax.numpy as jnp
from jax.experimental import pallas as pl


def kernel(x, edge_index, h, W1, b1, W2, b2, Wz, bz, Wr, br, Wc, bc):
    raise NotImplementedError("write your pallas kernel here")


# speedup vs baseline: 6.9230x; 6.9230x over previous
import functools

import jax
import jax.numpy as jnp
from jax import lax
from jax.experimental import pallas as pl
from jax.experimental.pallas import tpu as pltpu
from jax.experimental.pallas import tpu_sc as plsc

NC = 2
NS = 16
NW = NC * NS
C = 128


def _fill_const(ref, rows, d, value):
    vec = jnp.full((16,), value, jnp.float32)

    def row(r, carry):
        for q in range(d // 16):
            ref[r, pl.ds(q * 16, 16)] = vec
        return carry

    lax.fori_loop(0, rows, row, 0)


def _build_deg_kernel(np_, ch):
    mesh = plsc.VectorSubcoreMesh(core_axis_name="c", subcore_axis_name="s")
    tps = np_ // NS
    assert tps % 8 == 0

    @functools.partial(
        pl.kernel,
        out_type=jax.ShapeDtypeStruct((NC, np_, 128), jnp.float32),
        mesh=mesh,
        scratch_types=[
            pltpu.VMEM((ch, C), jnp.int32),
            pltpu.VMEM((C, 128), jnp.float32),
            pltpu.VMEM((8, 128), jnp.float32),
            pltpu.VMEM_SHARED((np_, 128), jnp.float32),
        ],
    )
    def deg_kernel(src_hbm, out_hbm, src_v, ones_v, zeros_v, acc_sh):
        cid = lax.axis_index("c")
        sid = lax.axis_index("s")
        wid = sid * NC + cid
        _fill_const(ones_v, C, 128, 1.0)
        _fill_const(zeros_v, 8, 128, 0.0)
        pltpu.sync_copy(src_hbm.at[wid], src_v)
        for z in range(tps // 8):
            pltpu.sync_copy(zeros_v, acc_sh.at[pl.ds(sid * tps + z * 8, 8)])
        plsc.subcore_barrier()

        def body(j, carry):
            pltpu.sync_copy(ones_v, acc_sh.at[src_v.at[j]], add=True)
            return carry

        lax.fori_loop(0, ch, body, 0)
        plsc.subcore_barrier()
        pltpu.sync_copy(acc_sh.at[pl.ds(sid * tps, tps)],
                        out_hbm.at[cid, pl.ds(sid * tps, tps)])

    return deg_kernel


def _build_scatter_kernel(np_, d, ch):
    mesh = plsc.VectorSubcoreMesh(core_axis_name="c", subcore_axis_name="s")
    tps = np_ // NS
    assert tps % 8 == 0

    @functools.partial(
        pl.kernel,
        out_type=(jax.ShapeDtypeStruct((NC, np_, d), jnp.float32),
                  jax.ShapeDtypeStruct((NC, np_, d), jnp.float32)),
        mesh=mesh,
        scratch_types=[
            pltpu.VMEM((ch, C), jnp.int32),
            pltpu.VMEM((ch, C), jnp.int32),
            pltpu.VMEM((C, d), jnp.float32),
            pltpu.VMEM((8, d), jnp.float32),
            pltpu.VMEM_SHARED((np_, d), jnp.float32),
        ],
    )
    def scat_kernel(t0_hbm, t1_hbm, src_hbm, dst_hbm, out0_hbm, out1_hbm,
                    src_v, dst_v, rows_v, zeros_v, acc_sh):
        cid = lax.axis_index("c")
        sid = lax.axis_index("s")
        wid = sid * NC + cid
        _fill_const(zeros_v, 8, d, 0.0)
        pltpu.sync_copy(src_hbm.at[wid], src_v)
        pltpu.sync_copy(dst_hbm.at[wid], dst_v)
        for t_hbm, out_hbm in ((t0_hbm, out0_hbm), (t1_hbm, out1_hbm)):
            for z in range(tps // 8):
                pltpu.sync_copy(zeros_v,
                                acc_sh.at[pl.ds(sid * tps + z * 8, 8)])
            plsc.subcore_barrier()

            def body(j, carry):
                pltpu.sync_copy(t_hbm.at[src_v.at[j]], rows_v)
                pltpu.sync_copy(rows_v, acc_sh.at[dst_v.at[j]], add=True)
                return carry

            lax.fori_loop(0, ch, body, 0)
            plsc.subcore_barrier()
            pltpu.sync_copy(acc_sh.at[pl.ds(sid * tps, tps)],
                            out_hbm.at[cid, pl.ds(sid * tps, tps)])
            plsc.subcore_barrier()

    return scat_kernel


def _dis_from_deg(degp):
    deg = degp[0, :, 0:1] + degp[1, :, 0:1]
    return jnp.where(deg > 0.0, lax.rsqrt(jnp.maximum(deg, 1.0)), 0.0)


def _build_scale_kernel(n, np_, d, nb):
    def body(degp_ref, x_ref, h_ref, xs_ref, hs_ref):
        dis = _dis_from_deg(degp_ref[...])
        xs_ref[...] = x_ref[...] * dis
        hs_ref[...] = h_ref[...] * dis

    return pl.pallas_call(
        body,
        grid=(n // nb,),
        in_specs=[
            pl.BlockSpec((NC, nb, 128), lambda i: (0, i, 0)),
            pl.BlockSpec((nb, d), lambda i: (i, 0)),
            pl.BlockSpec((nb, d), lambda i: (i, 0)),
        ],
        out_specs=[pl.BlockSpec((nb, d), lambda i: (i, 0))] * 2,
        out_shape=[jax.ShapeDtypeStruct((np_, d), jnp.float32)] * 2,
    )


def _build_mid_kernel(n, np_, d, nb):
    def body(degp_ref, p1x_ref, p1h_ref,
             tx1x_ref, tx1h_ref, t1xs_ref, t1hs_ref):
        dis = _dis_from_deg(degp_ref[...])
        px = p1x_ref[...]
        ph = p1h_ref[...]
        tx1x = -(dis * (px[0] + px[1]))
        tx1h = -(dis * (ph[0] + ph[1]))
        tx1x_ref[...] = tx1x
        tx1h_ref[...] = tx1h
        t1xs_ref[...] = dis * tx1x
        t1hs_ref[...] = dis * tx1h

    return pl.pallas_call(
        body,
        grid=(n // nb,),
        in_specs=[
            pl.BlockSpec((NC, nb, 128), lambda i: (0, i, 0)),
            pl.BlockSpec((NC, nb, d), lambda i: (0, i, 0)),
            pl.BlockSpec((NC, nb, d), lambda i: (0, i, 0)),
        ],
        out_specs=[pl.BlockSpec((nb, d), lambda i: (i, 0))] * 4,
        out_shape=[jax.ShapeDtypeStruct((n, d), jnp.float32)] * 2
        + [jax.ShapeDtypeStruct((np_, d), jnp.float32)] * 2,
    )


def _build_final_kernel(n, d, nb):
    def dot(a, b):
        return lax.dot(a, b, precision=lax.Precision.HIGHEST,
                       preferred_element_type=jnp.float32)

    def body(degp_ref, x_ref, h_ref, tx1x_ref, tx1h_ref, p2x_ref, p2h_ref,
             W1_ref, W2_ref, Wz_ref, Wr_ref, Wc_ref,
             b1_ref, b2_ref, bz_ref, br_ref, bc_ref, out_ref):
        dis = _dis_from_deg(degp_ref[...])
        xb = x_ref[...]
        hb = h_ref[...]
        px = p2x_ref[...]
        ph = p2h_ref[...]
        tx2x = -2.0 * (dis * (px[0] + px[1])) - xb
        tx2h = -2.0 * (dis * (ph[0] + ph[1])) - hb
        W1 = W1_ref[...]
        W2 = W2_ref[...]
        ic = (dot(xb, W1[0]) + dot(tx1x_ref[...], W1[1]) + dot(tx2x, W1[2])
              + b1_ref[...])
        hc = (dot(hb, W2[0]) + dot(tx1h_ref[...], W2[1]) + dot(tx2h, W2[2])
              + b2_ref[...])
        Wz = Wz_ref[...]
        Wr = Wr_ref[...]
        Wc = Wc_ref[...]
        z = jax.nn.sigmoid(dot(ic, Wz[:d]) + dot(hc, Wz[d:]) + bz_ref[...])
        r = jax.nn.sigmoid(dot(ic, Wr[:d]) + dot(hc, Wr[d:]) + br_ref[...])
        ht = jnp.tanh(dot(ic, Wc[:d]) + dot(r * hc, Wc[d:]) + bc_ref[...])
        out_ref[...] = z * hb + (1.0 - z) * ht

    full = lambda shape: pl.BlockSpec(shape, lambda i: tuple(0 for _ in shape))
    return pl.pallas_call(
        body,
        grid=(n // nb,),
        in_specs=[
            pl.BlockSpec((NC, nb, 128), lambda i: (0, i, 0)),
            pl.BlockSpec((nb, d), lambda i: (i, 0)),
            pl.BlockSpec((nb, d), lambda i: (i, 0)),
            pl.BlockSpec((nb, d), lambda i: (i, 0)),
            pl.BlockSpec((nb, d), lambda i: (i, 0)),
            pl.BlockSpec((NC, nb, d), lambda i: (0, i, 0)),
            pl.BlockSpec((NC, nb, d), lambda i: (0, i, 0)),
            full((3, d, d)),
            full((3, d, d)),
            full((2 * d, d)),
            full((2 * d, d)),
            full((2 * d, d)),
            full((1, d)),
            full((1, d)),
            full((1, d)),
            full((1, d)),
            full((1, d)),
        ],
        out_specs=pl.BlockSpec((nb, d), lambda i: (i, 0)),
        out_shape=jax.ShapeDtypeStruct((n, d), jnp.float32),
    )


def kernel(x, edge_index, h, W1, b1, W2, b2, Wz, bz, Wr, br, Wc, bc):
    n, d = x.shape
    e = edge_index.shape[1]
    assert h.shape == (n, d) and e % NW == 0
    ew = e // NW
    ew_p = -(-ew // C) * C
    ch = ew_p // C
    nb = 2000 if n % 2000 == 0 else 400
    np_ = -(-n // (NS * 8)) * (NS * 8)
    assert n % nb == 0 and np_ > n

    src_w = edge_index[0].reshape(NW, ew)
    dst_w = edge_index[1].reshape(NW, ew)
    pad = ew_p - ew
    pad_idx = jnp.full((NW, pad), np_ - 1, jnp.int32)
    src_r = jnp.concatenate([src_w, pad_idx], axis=1).reshape(NW, ch, C)
    dst_r = jnp.concatenate([dst_w, pad_idx], axis=1).reshape(NW, ch, C)

    deg_k = _build_deg_kernel(np_, ch)
    scat_k = _build_scatter_kernel(np_, d, ch)
    scale_k = _build_scale_kernel(n, np_, d, nb)
    mid_k = _build_mid_kernel(n, np_, d, nb)
    final_k = _build_final_kernel(n, d, nb)

    degp = deg_k(src_r)
    xs, hs = scale_k(degp, x, h)
    p1x, p1h = scat_k(xs, hs, src_r, dst_r)
    tx1x, tx1h, t1xs, t1hs = mid_k(degp, p1x, p1h)
    p2x, p2h = scat_k(t1xs, t1hs, src_r, dst_r)
    return final_k(
        degp, x, h, tx1x, tx1h, p2x, p2h,
        W1, W2, Wz, Wr, Wc,
        b1.reshape(1, d), b2.reshape(1, d), bz.reshape(1, d),
        br.reshape(1, d), bc.reshape(1, d),
    )


# speedup vs baseline: 8.0389x; 1.1612x over previous
import functools

import jax
import jax.numpy as jnp
from jax import lax
from jax.experimental import pallas as pl
from jax.experimental.pallas import tpu as pltpu
from jax.experimental.pallas import tpu_sc as plsc

NC = 2
NS = 16
NW = NC * NS
C = 128


def _fill_const(ref, rows, d, value):
    vec = jnp.full((16,), value, jnp.float32)

    def row(r, carry):
        for q in range(d // 16):
            ref[r, pl.ds(q * 16, 16)] = vec
        return carry

    lax.fori_loop(0, rows, row, 0)


def _build_deg_kernel(np_, ch):
    mesh = plsc.VectorSubcoreMesh(core_axis_name="c", subcore_axis_name="s")
    tps = np_ // NS
    assert tps % 8 == 0

    @functools.partial(
        pl.kernel,
        out_type=jax.ShapeDtypeStruct((NC, np_, 128), jnp.float32),
        mesh=mesh,
        scratch_types=[
            pltpu.VMEM((ch, C), jnp.int32),
            pltpu.VMEM((C, 128), jnp.float32),
            pltpu.VMEM((8, 128), jnp.float32),
            pltpu.VMEM_SHARED((np_, 128), jnp.float32),
        ],
    )
    def deg_kernel(src_hbm, out_hbm, src_v, ones_v, zeros_v, acc_sh):
        cid = lax.axis_index("c")
        sid = lax.axis_index("s")
        wid = sid * NC + cid
        _fill_const(ones_v, C, 128, 1.0)
        _fill_const(zeros_v, 8, 128, 0.0)
        pltpu.sync_copy(src_hbm.at[wid], src_v)
        for z in range(tps // 8):
            pltpu.sync_copy(zeros_v, acc_sh.at[pl.ds(sid * tps + z * 8, 8)])
        plsc.subcore_barrier()

        def body(j, carry):
            pltpu.sync_copy(ones_v, acc_sh.at[src_v.at[j]], add=True)
            return carry

        lax.fori_loop(0, ch, body, 0)
        plsc.subcore_barrier()
        pltpu.sync_copy(acc_sh.at[pl.ds(sid * tps, tps)],
                        out_hbm.at[cid, pl.ds(sid * tps, tps)])

    return deg_kernel


def _build_scatter_kernel(np_, d, ch):
    mesh = plsc.VectorSubcoreMesh(core_axis_name="c", subcore_axis_name="s")
    tps = np_ // NS
    assert tps % 8 == 0

    @functools.partial(
        pl.kernel,
        out_type=(jax.ShapeDtypeStruct((NC, np_, d), jnp.float32),
                  jax.ShapeDtypeStruct((NC, np_, d), jnp.float32)),
        mesh=mesh,
        scratch_types=[
            pltpu.VMEM((ch, C), jnp.int32),
            pltpu.VMEM((2, C), jnp.int32),
            pltpu.VMEM((C, d), jnp.float32),
            pltpu.VMEM((C, d), jnp.float32),
            pltpu.VMEM((8, d), jnp.float32),
            pltpu.VMEM_SHARED((np_, d), jnp.float32),
            pltpu.SemaphoreType.DMA,
            pltpu.SemaphoreType.DMA,
        ],
    )
    def scat_kernel(t0_hbm, t1_hbm, src_hbm, dst_hbm, out0_hbm, out1_hbm,
                    src_v, dring, rows0, rows1, zeros_v, acc_sh, sem0, sem1):
        cid = lax.axis_index("c")
        sid = lax.axis_index("s")
        wid = sid * NC + cid
        rows = (rows0, rows1)
        sems = (sem0, sem1)
        _fill_const(zeros_v, 8, d, 0.0)
        pltpu.sync_copy(src_hbm.at[wid], src_v)
        for t_hbm, out_hbm in ((t0_hbm, out0_hbm), (t1_hbm, out1_hbm)):
            for z in range(tps // 8):
                pltpu.sync_copy(zeros_v,
                                acc_sh.at[pl.ds(sid * tps + z * 8, 8)])
            plsc.subcore_barrier()

            def issue(j, b):
                pltpu.sync_copy(dst_hbm.at[wid, j], dring.at[b])
                pltpu.async_copy(t_hbm.at[src_v.at[j]], rows[b], sems[b])

            def finish(j, b):
                pltpu.make_async_copy(
                    t_hbm.at[src_v.at[j]], rows[b], sems[b]).wait()
                pltpu.sync_copy(rows[b], acc_sh.at[dring.at[b]], add=True)

            issue(0, 0)

            def pair(jj, carry):
                issue(2 * jj + 1, 1)
                finish(2 * jj, 0)
                issue(2 * jj + 2, 0)
                finish(2 * jj + 1, 1)
                return carry

            lax.fori_loop(0, (ch - 1) // 2, pair, 0)
            if ch % 2 == 1:
                finish(ch - 1, (ch - 1) % 2)
            else:
                issue(ch - 1, (ch - 1) % 2)
                finish(ch - 2, (ch - 2) % 2)
                finish(ch - 1, (ch - 1) % 2)
            plsc.subcore_barrier()
            pltpu.sync_copy(acc_sh.at[pl.ds(sid * tps, tps)],
                            out_hbm.at[cid, pl.ds(sid * tps, tps)])
            plsc.subcore_barrier()

    return scat_kernel


def _dis_from_deg(degp):
    deg = degp[0, :, 0:1] + degp[1, :, 0:1]
    return jnp.where(deg > 0.0, lax.rsqrt(jnp.maximum(deg, 1.0)), 0.0)


def _build_scale_kernel(n, np_, d, nb):
    def body(degp_ref, x_ref, h_ref, xs_ref, hs_ref):
        dis = _dis_from_deg(degp_ref[...])
        xs_ref[...] = x_ref[...] * dis
        hs_ref[...] = h_ref[...] * dis

    return pl.pallas_call(
        body,
        grid=(n // nb,),
        in_specs=[
            pl.BlockSpec((NC, nb, 128), lambda i: (0, i, 0)),
            pl.BlockSpec((nb, d), lambda i: (i, 0)),
            pl.BlockSpec((nb, d), lambda i: (i, 0)),
        ],
        out_specs=[pl.BlockSpec((nb, d), lambda i: (i, 0))] * 2,
        out_shape=[jax.ShapeDtypeStruct((np_, d), jnp.float32)] * 2,
    )


def _build_mid_kernel(n, np_, d, nb):
    def body(degp_ref, p1x_ref, p1h_ref,
             tx1x_ref, tx1h_ref, t1xs_ref, t1hs_ref):
        dis = _dis_from_deg(degp_ref[...])
        px = p1x_ref[...]
        ph = p1h_ref[...]
        tx1x = -(dis * (px[0] + px[1]))
        tx1h = -(dis * (ph[0] + ph[1]))
        tx1x_ref[...] = tx1x
        tx1h_ref[...] = tx1h
        t1xs_ref[...] = dis * tx1x
        t1hs_ref[...] = dis * tx1h

    return pl.pallas_call(
        body,
        grid=(n // nb,),
        in_specs=[
            pl.BlockSpec((NC, nb, 128), lambda i: (0, i, 0)),
            pl.BlockSpec((NC, nb, d), lambda i: (0, i, 0)),
            pl.BlockSpec((NC, nb, d), lambda i: (0, i, 0)),
        ],
        out_specs=[pl.BlockSpec((nb, d), lambda i: (i, 0))] * 4,
        out_shape=[jax.ShapeDtypeStruct((n, d), jnp.float32)] * 2
        + [jax.ShapeDtypeStruct((np_, d), jnp.float32)] * 2,
    )


def _build_final_kernel(n, d, nb):
    def dot(a, b):
        return lax.dot(a, b, precision=lax.Precision.HIGHEST,
                       preferred_element_type=jnp.float32)

    def body(degp_ref, x_ref, h_ref, tx1x_ref, tx1h_ref, p2x_ref, p2h_ref,
             W1_ref, W2_ref, Wz_ref, Wr_ref, Wc_ref,
             b1_ref, b2_ref, bz_ref, br_ref, bc_ref, out_ref):
        dis = _dis_from_deg(degp_ref[...])
        xb = x_ref[...]
        hb = h_ref[...]
        px = p2x_ref[...]
        ph = p2h_ref[...]
        tx2x = -2.0 * (dis * (px[0] + px[1])) - xb
        tx2h = -2.0 * (dis * (ph[0] + ph[1])) - hb
        W1 = W1_ref[...]
        W2 = W2_ref[...]
        ic = (dot(xb, W1[0]) + dot(tx1x_ref[...], W1[1]) + dot(tx2x, W1[2])
              + b1_ref[...])
        hc = (dot(hb, W2[0]) + dot(tx1h_ref[...], W2[1]) + dot(tx2h, W2[2])
              + b2_ref[...])
        Wz = Wz_ref[...]
        Wr = Wr_ref[...]
        Wc = Wc_ref[...]
        z = jax.nn.sigmoid(dot(ic, Wz[:d]) + dot(hc, Wz[d:]) + bz_ref[...])
        r = jax.nn.sigmoid(dot(ic, Wr[:d]) + dot(hc, Wr[d:]) + br_ref[...])
        ht = jnp.tanh(dot(ic, Wc[:d]) + dot(r * hc, Wc[d:]) + bc_ref[...])
        out_ref[...] = z * hb + (1.0 - z) * ht

    full = lambda shape: pl.BlockSpec(shape, lambda i: tuple(0 for _ in shape))
    return pl.pallas_call(
        body,
        grid=(n // nb,),
        in_specs=[
            pl.BlockSpec((NC, nb, 128), lambda i: (0, i, 0)),
            pl.BlockSpec((nb, d), lambda i: (i, 0)),
            pl.BlockSpec((nb, d), lambda i: (i, 0)),
            pl.BlockSpec((nb, d), lambda i: (i, 0)),
            pl.BlockSpec((nb, d), lambda i: (i, 0)),
            pl.BlockSpec((NC, nb, d), lambda i: (0, i, 0)),
            pl.BlockSpec((NC, nb, d), lambda i: (0, i, 0)),
            full((3, d, d)),
            full((3, d, d)),
            full((2 * d, d)),
            full((2 * d, d)),
            full((2 * d, d)),
            full((1, d)),
            full((1, d)),
            full((1, d)),
            full((1, d)),
            full((1, d)),
        ],
        out_specs=pl.BlockSpec((nb, d), lambda i: (i, 0)),
        out_shape=jax.ShapeDtypeStruct((n, d), jnp.float32),
    )


def kernel(x, edge_index, h, W1, b1, W2, b2, Wz, bz, Wr, br, Wc, bc):
    n, d = x.shape
    e = edge_index.shape[1]
    assert h.shape == (n, d) and e % NW == 0
    ew = e // NW
    ew_p = -(-ew // C) * C
    ch = ew_p // C
    nb = 2000 if n % 2000 == 0 else 400
    np_ = -(-n // (NS * 8)) * (NS * 8)
    assert n % nb == 0 and np_ > n

    src_w = edge_index[0].reshape(NW, ew)
    dst_w = edge_index[1].reshape(NW, ew)
    pad = ew_p - ew
    pad_idx = jnp.full((NW, pad), np_ - 1, jnp.int32)
    src_r = jnp.concatenate([src_w, pad_idx], axis=1).reshape(NW, ch, C)
    dst_r = jnp.concatenate([dst_w, pad_idx], axis=1).reshape(NW, ch, C)

    deg_k = _build_deg_kernel(np_, ch)
    scat_k = _build_scatter_kernel(np_, d, ch)
    scale_k = _build_scale_kernel(n, np_, d, nb)
    mid_k = _build_mid_kernel(n, np_, d, nb)
    final_k = _build_final_kernel(n, d, nb)

    degp = deg_k(src_r)
    xs, hs = scale_k(degp, x, h)
    p1x, p1h = scat_k(xs, hs, src_r, dst_r)
    tx1x, tx1h, t1xs, t1hs = mid_k(degp, p1x, p1h)
    p2x, p2h = scat_k(t1xs, t1hs, src_r, dst_r)
    return final_k(
        degp, x, h, tx1x, tx1h, p2x, p2h,
        W1, W2, Wz, Wr, Wc,
        b1.reshape(1, d), b2.reshape(1, d), bz.reshape(1, d),
        br.reshape(1, d), bc.reshape(1, d),
    )
